# baseline (speedup 1.0000x reference)
import functools

import jax
import jax.numpy as jnp
from jax import lax
from jax.experimental import pallas as pl
from jax.experimental.pallas import tpu as pltpu
from jax.experimental.pallas import tpu_sc as plsc

_NC = 2
_NS = 16
_NW = _NC * _NS
_LANES = 16


def _proj_body(x_ref, w_ref, p_ref):
    p_ref[...] = jnp.dot(x_ref[...], w_ref[...],
                         preferred_element_type=jnp.float32)


def _epilogue_body(dd_ref, db_ref, o_ref):
    d = dd_ref[...] + db_ref[0, 0]
    sp = jnp.maximum(d, 0.0) + jnp.log1p(jnp.exp(-jnp.abs(d)))
    lane = lax.broadcasted_iota(jnp.int32, d.shape, 1)
    o_ref[...] = jnp.where((lane & 1) == 0, -sp, d - sp)


def _make_sc_gather(n_nodes, n_edges):
    mesh = plsc.VectorSubcoreMesh(core_axis_name="c", subcore_axis_name="s")
    chunk = n_edges // _NW
    n_vec = -(-chunk // _LANES)

    @functools.partial(
        pl.kernel,
        out_type=jax.ShapeDtypeStruct((2 * n_edges,), jnp.float32),
        mesh=mesh,
        scratch_types=[
            pltpu.VMEM((2 * n_nodes,), jnp.float32),
            pltpu.VMEM((chunk,), jnp.int32),
            pltpu.VMEM((chunk,), jnp.int32),
            pltpu.VMEM((2 * chunk,), jnp.float32),
        ],
        compiler_params=pltpu.CompilerParams(needs_layout_passes=False),
    )
    def sc_gather(t_hbm, edges_hbm, out_hbm, t_v, src_v, dst_v, d_v):
        wid = lax.axis_index("s") * _NC + lax.axis_index("c")
        base = wid * chunk
        pltpu.sync_copy(t_hbm, t_v)
        pltpu.sync_copy(edges_hbm.at[pl.ds(base, chunk)], src_v)
        pltpu.sync_copy(edges_hbm.at[pl.ds(n_edges + base, chunk)], dst_v)
        lane2 = lax.iota(jnp.int32, _LANES) * 2

        def body(j, carry):
            off = jnp.minimum(j * _LANES, chunk - _LANES)
            idx_s = src_v[pl.ds(off, _LANES)]
            idx_d = dst_v[pl.ds(off, _LANES)]
            a = plsc.load_gather(t_v, [idx_s + idx_s])
            c = plsc.load_gather(t_v, [idx_d + idx_d + 1])
            d = a + c
            pos = lane2 + off * 2
            plsc.store_scatter(d_v, [pos], d)
            plsc.store_scatter(d_v, [pos + 1], d)
            return carry

        lax.fori_loop(0, n_vec, body, 0, unroll=4)
        pltpu.sync_copy(d_v, out_hbm.at[pl.ds(2 * base, 2 * chunk)])

    return sc_gather


def kernel(node_features_after_gcn, edges, W, b):
    x = node_features_after_gcn
    n_nodes, hidden = x.shape
    n_edges = edges.shape[1]

    wd = W[1] - W[0]
    wstack = jnp.stack([wd[:hidden], wd[hidden:]], axis=1)
    db = (b[1] - b[0]).reshape(1, 1)

    n_blocks = 5
    rows = n_nodes // n_blocks
    proj = pl.pallas_call(
        _proj_body,
        grid=(n_blocks,),
        in_specs=[
            pl.BlockSpec((rows, hidden), lambda i: (i, 0)),
            pl.BlockSpec((hidden, 2), lambda i: (0, 0)),
        ],
        out_specs=pl.BlockSpec((rows, 2), lambda i: (i, 0)),
        out_shape=jax.ShapeDtypeStruct((n_nodes, 2), jnp.float32),
    )(x, wstack)

    dd = _make_sc_gather(n_nodes, n_edges)(proj.reshape(2 * n_nodes),
                                           edges.reshape(2 * n_edges))

    dd2 = dd.reshape(n_edges // 128, 256)
    out = pl.pallas_call(
        _epilogue_body,
        in_specs=[
            pl.BlockSpec(memory_space=pltpu.VMEM),
            pl.BlockSpec(memory_space=pltpu.SMEM),
        ],
        out_shape=jax.ShapeDtypeStruct(dd2.shape, jnp.float32),
    )(dd2, db)

    return out.reshape(n_edges, 2)


# speedup vs baseline: 3.3638x; 3.3638x over previous
import functools

import jax
import jax.numpy as jnp
from jax import lax
from jax.experimental import pallas as pl
from jax.experimental.pallas import tpu as pltpu
from jax.experimental.pallas import tpu_sc as plsc

_NC = 2
_NS = 16
_NW = _NC * _NS
_LANES = 16


def _proj_body(x_ref, w_ref, p_ref):
    p_ref[...] = jnp.dot(x_ref[...], w_ref[...],
                         preferred_element_type=jnp.float32)


def _epilogue_body(d_ref, db_ref, o0_ref, o1_ref):
    d = d_ref[...] + db_ref[0, 0]
    sp = jnp.maximum(d, 0.0) + jnp.log1p(jnp.exp(-jnp.abs(d)))
    o0_ref[...] = -sp
    o1_ref[...] = d - sp


def _make_sc_gather(n_nodes, n_edges):
    mesh = plsc.VectorSubcoreMesh(core_axis_name="c", subcore_axis_name="s")
    chunk = n_edges // _NW
    n_vec = -(-chunk // _LANES)

    @functools.partial(
        pl.kernel,
        out_type=jax.ShapeDtypeStruct((n_edges,), jnp.float32),
        mesh=mesh,
        scratch_types=[
            pltpu.VMEM((2 * n_nodes,), jnp.float32),
            pltpu.VMEM((chunk,), jnp.int32),
            pltpu.VMEM((chunk,), jnp.int32),
            pltpu.VMEM((chunk,), jnp.float32),
        ],
        compiler_params=pltpu.CompilerParams(needs_layout_passes=False),
    )
    def sc_gather(t_hbm, edges_hbm, out_hbm, t_v, src_v, dst_v, d_v):
        wid = lax.axis_index("s") * _NC + lax.axis_index("c")
        base = wid * chunk
        pltpu.sync_copy(t_hbm, t_v)
        pltpu.sync_copy(edges_hbm.at[pl.ds(base, chunk)], src_v)
        pltpu.sync_copy(edges_hbm.at[pl.ds(n_edges + base, chunk)], dst_v)
        def body(j, carry):
            off = jnp.minimum(j * _LANES, chunk - _LANES)
            idx_s = src_v[pl.ds(off, _LANES)]
            idx_d = dst_v[pl.ds(off, _LANES)]
            a = plsc.load_gather(t_v, [idx_s + idx_s])
            c = plsc.load_gather(t_v, [idx_d + idx_d + 1])
            d_v[pl.ds(off, _LANES)] = a + c
            return carry

        lax.fori_loop(0, n_vec, body, 0, unroll=4)
        pltpu.sync_copy(d_v, out_hbm.at[pl.ds(base, chunk)])

    return sc_gather


def kernel(node_features_after_gcn, edges, W, b):
    x = node_features_after_gcn
    n_nodes, hidden = x.shape
    n_edges = edges.shape[1]

    wd = W[1] - W[0]
    wstack = jnp.stack([wd[:hidden], wd[hidden:]], axis=1)
    db = (b[1] - b[0]).reshape(1, 1)

    n_blocks = 5
    rows = n_nodes // n_blocks
    proj = pl.pallas_call(
        _proj_body,
        grid=(n_blocks,),
        in_specs=[
            pl.BlockSpec((rows, hidden), lambda i: (i, 0)),
            pl.BlockSpec((hidden, 2), lambda i: (0, 0)),
        ],
        out_specs=pl.BlockSpec((rows, 2), lambda i: (i, 0)),
        out_shape=jax.ShapeDtypeStruct((n_nodes, 2), jnp.float32),
    )(x, wstack)

    d = _make_sc_gather(n_nodes, n_edges)(proj.reshape(2 * n_nodes),
                                          edges.reshape(2 * n_edges))

    d2 = d.reshape(n_edges // 128, 128)
    o0, o1 = pl.pallas_call(
        _epilogue_body,
        in_specs=[
            pl.BlockSpec(memory_space=pltpu.VMEM),
            pl.BlockSpec(memory_space=pltpu.SMEM),
        ],
        out_shape=[
            jax.ShapeDtypeStruct(d2.shape, jnp.float32),
            jax.ShapeDtypeStruct(d2.shape, jnp.float32),
        ],
    )(d2, db)

    return jnp.stack([o0.reshape(n_edges), o1.reshape(n_edges)], axis=1)
